# baseline (speedup 1.0000x reference)
import functools

import jax
import jax.numpy as jnp
from jax import lax
from jax.experimental import pallas as pl
from jax.experimental.pallas import tpu as pltpu
from jax.experimental.pallas import tpu_sc as plsc

B, L, V, D = 1024, 200, 100000, 300
DP = 512
DPK = 256
C_OUT, K, H, NCLS = 128, 5, 20, 10
L_OUT = L - K + 1

NW = 32
NCHUNK = 2
N_TOK = B * L // NCHUNK
PER_W = N_TOK // NW
CH = 80
N_CH = PER_W // CH


def _sc_gather_body(idx_hbm, table_hbm, out_hbm, idx_v, buf0, buf1, sem0, sem1):
    wid = lax.axis_index("s") * 2 + lax.axis_index("c")
    base = wid * PER_W
    pltpu.sync_copy(idx_hbm.at[wid], idx_v)

    pltpu.make_async_copy(table_hbm.at[idx_v.at[0]], buf0, sem0).start()

    def pair(jj, carry):
        j0 = 2 * jj
        pltpu.make_async_copy(table_hbm.at[idx_v.at[j0]], buf0, sem0).wait()
        pltpu.make_async_copy(table_hbm.at[idx_v.at[j0 + 1]], buf1, sem1).start()
        pltpu.sync_copy(buf0, out_hbm.at[pl.ds(base + j0 * CH, CH)])
        pltpu.make_async_copy(table_hbm.at[idx_v.at[j0 + 1]], buf1, sem1).wait()

        @pl.when(jj + 1 < N_CH // 2)
        def _():
            pltpu.make_async_copy(table_hbm.at[idx_v.at[j0 + 2]], buf0, sem0).start()

        pltpu.sync_copy(buf1, out_hbm.at[pl.ds(base + (j0 + 1) * CH, CH)])
        return carry

    lax.fori_loop(0, N_CH // 2, pair, 0)


@functools.cache
def _sc_gather():
    return pl.kernel(
        _sc_gather_body,
        mesh=plsc.VectorSubcoreMesh(core_axis_name="c", subcore_axis_name="s"),
        out_type=jax.ShapeDtypeStruct((N_TOK, DPK), jnp.int32),
        scratch_types=[
            pltpu.VMEM((N_CH, CH), jnp.int32),
            pltpu.VMEM((CH, DPK), jnp.int32),
            pltpu.VMEM((CH, DPK), jnp.int32),
            pltpu.SemaphoreType.DMA,
            pltpu.SemaphoreType.DMA,
        ],
    )


PAD_ROWS = 2000


def _bf16_bits(v):
    u = lax.bitcast_convert_type(v, jnp.uint32)
    return (u + 0x7FFF + ((u >> 16) & 1)) >> 16


def _pack_body(t_ref, o_ref):
    x = t_ref[...]
    xp = jnp.pad(x, ((0, 0), (0, DP - D)))
    lo = _bf16_bits(xp[:, :DPK])
    hi = _bf16_bits(xp[:, DPK:])
    o_ref[...] = lax.bitcast_convert_type(lo | (hi << 16), jnp.int32)


def _pack_table(table):
    return pl.pallas_call(
        _pack_body,
        grid=(V // PAD_ROWS,),
        in_specs=[pl.BlockSpec((PAD_ROWS, D), lambda i: (i, 0))],
        out_specs=pl.BlockSpec((PAD_ROWS, DPK), lambda i: (i, 0)),
        out_shape=jax.ShapeDtypeStruct((V, DPK), jnp.int32),
    )(table)


BB = 8


def _tc_body(emb_ref, wcat_ref, cb_ref, w1_ref, b1_ref, w2_ref, b2_ref, out_ref):
    u = lax.bitcast_convert_type(emb_ref[...], jnp.uint32)
    f_lo = lax.bitcast_convert_type(u << 16, jnp.float32)
    f_hi = lax.bitcast_convert_type(u & jnp.uint32(0xFFFF0000), jnp.float32)
    q = (jnp.dot(f_lo, wcat_ref[:DPK], preferred_element_type=jnp.float32)
         + jnp.dot(f_hi[:, :D - DPK], wcat_ref[DPK:D],
                   preferred_element_type=jnp.float32))
    q = q.reshape(BB, L, K * C_OUT)
    acc = q[:, 0:L_OUT, 0:C_OUT]
    for k in range(1, K):
        acc = acc + q[:, k:k + L_OUT, k * C_OUT:(k + 1) * C_OUT]
    h = jnp.maximum(acc + cb_ref[...], 0.0)
    p = jnp.max(h, axis=1)
    z1 = lax.dot_general(p, w1_ref[...], (((1,), (1,)), ((), ())),
                         preferred_element_type=jnp.float32) + b1_ref[...]
    z1 = jnp.maximum(z1, 0.0)
    z2 = lax.dot_general(z1, w2_ref[...], (((1,), (1,)), ((), ())),
                         preferred_element_type=jnp.float32) + b2_ref[...]
    m = jnp.max(z2, axis=1, keepdims=True)
    ez = jnp.exp(z2 - m)
    out_ref[...] = ez / jnp.sum(ez, axis=1, keepdims=True)


def _tc_head(emb, wcat, cb, w1, b1, w2, b2):
    return pl.pallas_call(
        _tc_body,
        grid=(B // NCHUNK // BB,),
        in_specs=[
            pl.BlockSpec((BB * L, DPK), lambda i: (i, 0)),
            pl.BlockSpec((D, K * C_OUT), lambda i: (0, 0)),
            pl.BlockSpec((1, C_OUT), lambda i: (0, 0)),
            pl.BlockSpec((H, C_OUT), lambda i: (0, 0)),
            pl.BlockSpec((1, H), lambda i: (0, 0)),
            pl.BlockSpec((NCLS, H), lambda i: (0, 0)),
            pl.BlockSpec((1, NCLS), lambda i: (0, 0)),
        ],
        out_specs=pl.BlockSpec((BB, NCLS), lambda i: (i, 0)),
        out_shape=jax.ShapeDtypeStruct((B // NCHUNK, NCLS), jnp.float32),
    )(emb, wcat, cb, w1, b1, w2, b2)


def kernel(x, table, conv_w, conv_b, W1, b1, W2, b2):
    idx = x.astype(jnp.int32).reshape(NCHUNK, NW, N_CH, CH)
    table_p = _pack_table(table)
    wcat = jnp.transpose(conv_w, (1, 2, 0)).reshape(D, K * C_OUT)
    cb = conv_b.reshape(1, C_OUT)
    b1r = b1.reshape(1, H)
    b2r = b2.reshape(1, NCLS)
    gather = _sc_gather()
    outs = []
    for c in range(NCHUNK):
        emb_c = gather(idx[c], table_p)
        outs.append(_tc_head(emb_c, wcat, cb, W1, b1r, W2, b2r))
    return jnp.concatenate(outs, axis=0)


# speedup vs baseline: 1.0250x; 1.0250x over previous
import functools

import jax
import jax.numpy as jnp
from jax import lax
from jax.experimental import pallas as pl
from jax.experimental.pallas import tpu as pltpu
from jax.experimental.pallas import tpu_sc as plsc

B, L, V, D = 1024, 200, 100000, 300
DP = 512
DPK = 256
C_OUT, K, H, NCLS = 128, 5, 20, 10
L_OUT = L - K + 1

NW = 32
NCHUNK = 4
N_TOK = B * L // NCHUNK
PER_W = N_TOK // NW
CH = 80
N_CH = PER_W // CH


def _sc_gather_body(idx_hbm, table_hbm, out_hbm, idx_v,
                    b0, b1, b2, b3, sg0, sg1, sg2, sg3, sw0, sw1, sw2, sw3):
    bufs = (b0, b1, b2, b3)
    sgs = (sg0, sg1, sg2, sg3)
    sws = (sw0, sw1, sw2, sw3)
    wid = lax.axis_index("s") * 2 + lax.axis_index("c")
    base = wid * PER_W
    pltpu.sync_copy(idx_hbm.at[wid], idx_v)

    def gcopy(j, i):
        return pltpu.make_async_copy(table_hbm.at[idx_v.at[j]], bufs[i], sgs[i])

    def wcopy(j, i):
        return pltpu.make_async_copy(bufs[i], out_hbm.at[pl.ds(base + j * CH, CH)],
                                     sws[i])

    for i in range(3):
        gcopy(i, i).start()
    gcopy(0, 0).wait(); wcopy(0, 0).start(); gcopy(3, 3).start()
    for j in range(1, 4):
        i = j % 4
        gcopy(j, i).wait()
        wcopy(j, i).start()
        wcopy(j - 1, i - 1).wait()
        gcopy(j + 3, i - 1).start()

    def grp(jj, carry):
        j0 = jj * 4
        for i in range(4):
            j = j0 + i
            gcopy(j, i).wait()
            wcopy(j, i).start()

            @pl.when(j + 3 < N_CH)
            def _():
                wcopy(j - 1, (i + 3) % 4).wait()
                gcopy(j + 3, (i + 3) % 4).start()

        return carry

    lax.fori_loop(1, N_CH // 4, grp, 0)

    for i in range(4):
        wcopy(0, i).wait()


@functools.cache
def _sc_gather():
    return pl.kernel(
        _sc_gather_body,
        mesh=plsc.VectorSubcoreMesh(core_axis_name="c", subcore_axis_name="s"),
        out_type=jax.ShapeDtypeStruct((N_TOK, DPK), jnp.int32),
        scratch_types=(
            [pltpu.VMEM((N_CH, CH), jnp.int32)]
            + [pltpu.VMEM((CH, DPK), jnp.int32)] * 4
            + [pltpu.SemaphoreType.DMA] * 8
        ),
    )


PAD_ROWS = 2000


def _bf16_bits(v):
    u = lax.bitcast_convert_type(v, jnp.uint32)
    return (u + 0x7FFF + ((u >> 16) & 1)) >> 16


def _pack_body(t_ref, o_ref):
    x = t_ref[...]
    xp = jnp.pad(x, ((0, 0), (0, DP - D)))
    lo = _bf16_bits(xp[:, :DPK])
    hi = _bf16_bits(xp[:, DPK:])
    o_ref[...] = lax.bitcast_convert_type(lo | (hi << 16), jnp.int32)


def _pack_table(table):
    return pl.pallas_call(
        _pack_body,
        grid=(V // PAD_ROWS,),
        in_specs=[pl.BlockSpec((PAD_ROWS, D), lambda i: (i, 0))],
        out_specs=pl.BlockSpec((PAD_ROWS, DPK), lambda i: (i, 0)),
        out_shape=jax.ShapeDtypeStruct((V, DPK), jnp.int32),
    )(table)


BB = 8


def _tc_body(emb_ref, wcat_ref, cb_ref, w1_ref, b1_ref, w2_ref, b2_ref, out_ref):
    u = lax.bitcast_convert_type(emb_ref[...], jnp.uint32)
    f_lo = lax.bitcast_convert_type(u << 16, jnp.float32)
    f_hi = lax.bitcast_convert_type(u & jnp.uint32(0xFFFF0000), jnp.float32)
    q = (jnp.dot(f_lo, wcat_ref[:DPK], preferred_element_type=jnp.float32)
         + jnp.dot(f_hi[:, :D - DPK], wcat_ref[DPK:D],
                   preferred_element_type=jnp.float32))
    q = q.reshape(BB, L, K * C_OUT)
    acc = q[:, 0:L_OUT, 0:C_OUT]
    for k in range(1, K):
        acc = acc + q[:, k:k + L_OUT, k * C_OUT:(k + 1) * C_OUT]
    h = jnp.maximum(acc + cb_ref[...], 0.0)
    p = jnp.max(h, axis=1)
    z1 = lax.dot_general(p, w1_ref[...], (((1,), (1,)), ((), ())),
                         preferred_element_type=jnp.float32) + b1_ref[...]
    z1 = jnp.maximum(z1, 0.0)
    z2 = lax.dot_general(z1, w2_ref[...], (((1,), (1,)), ((), ())),
                         preferred_element_type=jnp.float32) + b2_ref[...]
    m = jnp.max(z2, axis=1, keepdims=True)
    ez = jnp.exp(z2 - m)
    out_ref[...] = ez / jnp.sum(ez, axis=1, keepdims=True)


def _tc_head(emb, wcat, cb, w1, b1, w2, b2):
    return pl.pallas_call(
        _tc_body,
        grid=(B // NCHUNK // BB,),
        in_specs=[
            pl.BlockSpec((BB * L, DPK), lambda i: (i, 0)),
            pl.BlockSpec((D, K * C_OUT), lambda i: (0, 0)),
            pl.BlockSpec((1, C_OUT), lambda i: (0, 0)),
            pl.BlockSpec((H, C_OUT), lambda i: (0, 0)),
            pl.BlockSpec((1, H), lambda i: (0, 0)),
            pl.BlockSpec((NCLS, H), lambda i: (0, 0)),
            pl.BlockSpec((1, NCLS), lambda i: (0, 0)),
        ],
        out_specs=pl.BlockSpec((BB, NCLS), lambda i: (i, 0)),
        out_shape=jax.ShapeDtypeStruct((B // NCHUNK, NCLS), jnp.float32),
    )(emb, wcat, cb, w1, b1, w2, b2)


def kernel(x, table, conv_w, conv_b, W1, b1, W2, b2):
    idx = x.astype(jnp.int32).reshape(NCHUNK, NW, N_CH, CH)
    table_p = _pack_table(table)
    wcat = jnp.transpose(conv_w, (1, 2, 0)).reshape(D, K * C_OUT)
    cb = conv_b.reshape(1, C_OUT)
    b1r = b1.reshape(1, H)
    b2r = b2.reshape(1, NCLS)
    gather = _sc_gather()
    outs = []
    for c in range(NCHUNK):
        emb_c = gather(idx[c], table_p)
        outs.append(_tc_head(emb_c, wcat, cb, W1, b1r, W2, b2r))
    return jnp.concatenate(outs, axis=0)


# speedup vs baseline: 1.1240x; 1.0966x over previous
import functools

import jax
import jax.numpy as jnp
from jax import lax
from jax.experimental import pallas as pl
from jax.experimental.pallas import tpu as pltpu
from jax.experimental.pallas import tpu_sc as plsc

B, L, V, D = 1024, 200, 100000, 300
DP = 512
DPK = 256
C_OUT, K, H, NCLS = 128, 5, 20, 10
L_OUT = L - K + 1

NW = 32
NCHUNK = 4
N_TOK = B * L // NCHUNK
PER_W = N_TOK // NW
CH = 80
N_CH = PER_W // CH


def _sc_gather_body(idx_hbm, table_hbm, out_hbm, idx_v, buf0, buf1, sem0, sem1):
    wid = lax.axis_index("s") * 2 + lax.axis_index("c")
    base = wid * PER_W
    pltpu.sync_copy(idx_hbm.at[wid], idx_v)

    pltpu.make_async_copy(table_hbm.at[idx_v.at[0]], buf0, sem0).start()

    def pair(jj, carry):
        j0 = 2 * jj
        pltpu.make_async_copy(table_hbm.at[idx_v.at[j0]], buf0, sem0).wait()
        pltpu.make_async_copy(table_hbm.at[idx_v.at[j0 + 1]], buf1, sem1).start()
        pltpu.sync_copy(buf0, out_hbm.at[pl.ds(base + j0 * CH, CH)])
        pltpu.make_async_copy(table_hbm.at[idx_v.at[j0 + 1]], buf1, sem1).wait()

        @pl.when(jj + 1 < N_CH // 2)
        def _():
            pltpu.make_async_copy(table_hbm.at[idx_v.at[j0 + 2]], buf0, sem0).start()

        pltpu.sync_copy(buf1, out_hbm.at[pl.ds(base + (j0 + 1) * CH, CH)])
        return carry

    lax.fori_loop(0, N_CH // 2, pair, 0)


@functools.cache
def _sc_gather():
    return pl.kernel(
        _sc_gather_body,
        mesh=plsc.VectorSubcoreMesh(core_axis_name="c", subcore_axis_name="s"),
        out_type=jax.ShapeDtypeStruct((N_TOK, DPK), jnp.int32),
        scratch_types=[
            pltpu.VMEM((N_CH, CH), jnp.int32),
            pltpu.VMEM((CH, DPK), jnp.int32),
            pltpu.VMEM((CH, DPK), jnp.int32),
            pltpu.SemaphoreType.DMA,
            pltpu.SemaphoreType.DMA,
        ],
    )


PAD_ROWS = 2000


def _bf16_bits(v):
    u = lax.bitcast_convert_type(v, jnp.uint32)
    return (u + 0x7FFF + ((u >> 16) & 1)) >> 16


def _pack_body(t_ref, o_ref):
    x = t_ref[...]
    xp = jnp.pad(x, ((0, 0), (0, DP - D)))
    lo = _bf16_bits(xp[:, :DPK])
    hi = _bf16_bits(xp[:, DPK:])
    o_ref[...] = lax.bitcast_convert_type(lo | (hi << 16), jnp.int32)


def _pack_table(table):
    return pl.pallas_call(
        _pack_body,
        grid=(V // PAD_ROWS,),
        in_specs=[pl.BlockSpec((PAD_ROWS, D), lambda i: (i, 0))],
        out_specs=pl.BlockSpec((PAD_ROWS, DPK), lambda i: (i, 0)),
        out_shape=jax.ShapeDtypeStruct((V, DPK), jnp.int32),
    )(table)


BB = 16


def _tc_body(emb_ref, wcat_ref, cb_ref, w1_ref, b1_ref, w2_ref, b2_ref, out_ref):
    u = lax.bitcast_convert_type(emb_ref[...], jnp.uint32)
    f_lo = lax.bitcast_convert_type(u << 16, jnp.float32)
    f_hi = lax.bitcast_convert_type(u & jnp.uint32(0xFFFF0000), jnp.float32)
    q = (jnp.dot(f_lo, wcat_ref[:DPK], preferred_element_type=jnp.float32)
         + jnp.dot(f_hi[:, :D - DPK], wcat_ref[DPK:D],
                   preferred_element_type=jnp.float32))
    q = q.reshape(BB, L, K * C_OUT)
    acc = q[:, 0:L_OUT, 0:C_OUT]
    for k in range(1, K):
        acc = acc + q[:, k:k + L_OUT, k * C_OUT:(k + 1) * C_OUT]
    h = jnp.maximum(acc + cb_ref[...], 0.0)
    p = jnp.max(h, axis=1)
    z1 = lax.dot_general(p, w1_ref[...], (((1,), (1,)), ((), ())),
                         preferred_element_type=jnp.float32) + b1_ref[...]
    z1 = jnp.maximum(z1, 0.0)
    z2 = lax.dot_general(z1, w2_ref[...], (((1,), (1,)), ((), ())),
                         preferred_element_type=jnp.float32) + b2_ref[...]
    m = jnp.max(z2, axis=1, keepdims=True)
    ez = jnp.exp(z2 - m)
    out_ref[...] = ez / jnp.sum(ez, axis=1, keepdims=True)


def _tc_head(emb, wcat, cb, w1, b1, w2, b2):
    return pl.pallas_call(
        _tc_body,
        grid=(B // NCHUNK // BB,),
        in_specs=[
            pl.BlockSpec((BB * L, DPK), lambda i: (i, 0)),
            pl.BlockSpec((D, K * C_OUT), lambda i: (0, 0)),
            pl.BlockSpec((1, C_OUT), lambda i: (0, 0)),
            pl.BlockSpec((H, C_OUT), lambda i: (0, 0)),
            pl.BlockSpec((1, H), lambda i: (0, 0)),
            pl.BlockSpec((NCLS, H), lambda i: (0, 0)),
            pl.BlockSpec((1, NCLS), lambda i: (0, 0)),
        ],
        out_specs=pl.BlockSpec((BB, NCLS), lambda i: (i, 0)),
        out_shape=jax.ShapeDtypeStruct((B // NCHUNK, NCLS), jnp.float32),
    )(emb, wcat, cb, w1, b1, w2, b2)


def kernel(x, table, conv_w, conv_b, W1, b1, W2, b2):
    idx = x.astype(jnp.int32).reshape(NCHUNK, NW, N_CH, CH)
    table_p = _pack_table(table)
    wcat = jnp.transpose(conv_w, (1, 2, 0)).reshape(D, K * C_OUT)
    cb = conv_b.reshape(1, C_OUT)
    b1r = b1.reshape(1, H)
    b2r = b2.reshape(1, NCLS)
    gather = _sc_gather()
    outs = []
    for c in range(NCHUNK):
        emb_c = gather(idx[c], table_p)
        outs.append(_tc_head(emb_c, wcat, cb, W1, b1r, W2, b2r))
    return jnp.concatenate(outs, axis=0)


# speedup vs baseline: 1.1425x; 1.0165x over previous
import functools

import jax
import jax.numpy as jnp
from jax import lax
from jax.experimental import pallas as pl
from jax.experimental.pallas import tpu as pltpu
from jax.experimental.pallas import tpu_sc as plsc

B, L, V, D = 1024, 200, 100000, 300
DP = 512
DPK = 256
C_OUT, K, H, NCLS = 128, 5, 20, 10
L_OUT = L - K + 1

NW = 32
NCHUNK = 4
N_TOK = B * L // NCHUNK
PER_W = N_TOK // NW
CH = 80
N_CH = PER_W // CH


def _sc_gather_body(idx_hbm, table_hbm, out_hbm, idx_v, buf0, buf1, sem0, sem1):
    wid = lax.axis_index("s") * 2 + lax.axis_index("c")
    base = wid * PER_W
    pltpu.sync_copy(idx_hbm.at[wid], idx_v)

    pltpu.make_async_copy(table_hbm.at[idx_v.at[0]], buf0, sem0).start()

    def pair(jj, carry):
        j0 = 2 * jj
        pltpu.make_async_copy(table_hbm.at[idx_v.at[j0]], buf0, sem0).wait()
        pltpu.make_async_copy(table_hbm.at[idx_v.at[j0 + 1]], buf1, sem1).start()
        pltpu.sync_copy(buf0, out_hbm.at[pl.ds(base + j0 * CH, CH)])
        pltpu.make_async_copy(table_hbm.at[idx_v.at[j0 + 1]], buf1, sem1).wait()

        @pl.when(jj + 1 < N_CH // 2)
        def _():
            pltpu.make_async_copy(table_hbm.at[idx_v.at[j0 + 2]], buf0, sem0).start()

        pltpu.sync_copy(buf1, out_hbm.at[pl.ds(base + (j0 + 1) * CH, CH)])
        return carry

    lax.fori_loop(0, N_CH // 2, pair, 0)


@functools.cache
def _sc_gather():
    return pl.kernel(
        _sc_gather_body,
        mesh=plsc.VectorSubcoreMesh(core_axis_name="c", subcore_axis_name="s"),
        out_type=jax.ShapeDtypeStruct((N_TOK, DPK), jnp.int32),
        scratch_types=[
            pltpu.VMEM((N_CH, CH), jnp.int32),
            pltpu.VMEM((CH, DPK), jnp.int32),
            pltpu.VMEM((CH, DPK), jnp.int32),
            pltpu.SemaphoreType.DMA,
            pltpu.SemaphoreType.DMA,
        ],
    )


PAD_ROWS = 2000


def _bf16_bits(v):
    u = lax.bitcast_convert_type(v, jnp.uint32)
    return (u + 0x7FFF + ((u >> 16) & 1)) >> 16


def _pack_body(t_ref, o_ref):
    x = t_ref[...]
    xp = jnp.pad(x, ((0, 0), (0, DP - D)))
    lo = _bf16_bits(xp[:, :DPK])
    hi = _bf16_bits(xp[:, DPK:])
    o_ref[...] = lax.bitcast_convert_type(lo | (hi << 16), jnp.int32)


def _pack_table(table):
    return pl.pallas_call(
        _pack_body,
        grid=(V // PAD_ROWS,),
        in_specs=[pl.BlockSpec((PAD_ROWS, D), lambda i: (i, 0))],
        out_specs=pl.BlockSpec((PAD_ROWS, DPK), lambda i: (i, 0)),
        out_shape=jax.ShapeDtypeStruct((V, DPK), jnp.int32),
    )(table)


BB = 32


def _tc_body(emb_ref, wcat_ref, cb_ref, w1_ref, b1_ref, w2_ref, b2_ref, out_ref):
    u = lax.bitcast_convert_type(emb_ref[...], jnp.uint32)
    f_lo = lax.bitcast_convert_type(u << 16, jnp.float32)
    f_hi = lax.bitcast_convert_type(u & jnp.uint32(0xFFFF0000), jnp.float32)
    q = (jnp.dot(f_lo, wcat_ref[:DPK], preferred_element_type=jnp.float32)
         + jnp.dot(f_hi[:, :D - DPK], wcat_ref[DPK:D],
                   preferred_element_type=jnp.float32))
    q = q.reshape(BB, L, K * C_OUT)
    acc = q[:, 0:L_OUT, 0:C_OUT]
    for k in range(1, K):
        acc = acc + q[:, k:k + L_OUT, k * C_OUT:(k + 1) * C_OUT]
    h = jnp.maximum(acc + cb_ref[...], 0.0)
    p = jnp.max(h, axis=1)
    z1 = lax.dot_general(p, w1_ref[...], (((1,), (1,)), ((), ())),
                         preferred_element_type=jnp.float32) + b1_ref[...]
    z1 = jnp.maximum(z1, 0.0)
    z2 = lax.dot_general(z1, w2_ref[...], (((1,), (1,)), ((), ())),
                         preferred_element_type=jnp.float32) + b2_ref[...]
    m = jnp.max(z2, axis=1, keepdims=True)
    ez = jnp.exp(z2 - m)
    out_ref[...] = ez / jnp.sum(ez, axis=1, keepdims=True)


def _tc_head(emb, wcat, cb, w1, b1, w2, b2):
    return pl.pallas_call(
        _tc_body,
        grid=(B // NCHUNK // BB,),
        in_specs=[
            pl.BlockSpec((BB * L, DPK), lambda i: (i, 0)),
            pl.BlockSpec((D, K * C_OUT), lambda i: (0, 0)),
            pl.BlockSpec((1, C_OUT), lambda i: (0, 0)),
            pl.BlockSpec((H, C_OUT), lambda i: (0, 0)),
            pl.BlockSpec((1, H), lambda i: (0, 0)),
            pl.BlockSpec((NCLS, H), lambda i: (0, 0)),
            pl.BlockSpec((1, NCLS), lambda i: (0, 0)),
        ],
        out_specs=pl.BlockSpec((BB, NCLS), lambda i: (i, 0)),
        out_shape=jax.ShapeDtypeStruct((B // NCHUNK, NCLS), jnp.float32),
    )(emb, wcat, cb, w1, b1, w2, b2)


def kernel(x, table, conv_w, conv_b, W1, b1, W2, b2):
    idx = x.astype(jnp.int32).reshape(NCHUNK, NW, N_CH, CH)
    table_p = _pack_table(table)
    wcat = jnp.transpose(conv_w, (1, 2, 0)).reshape(D, K * C_OUT)
    cb = conv_b.reshape(1, C_OUT)
    b1r = b1.reshape(1, H)
    b2r = b2.reshape(1, NCLS)
    gather = _sc_gather()
    outs = []
    for c in range(NCHUNK):
        emb_c = gather(idx[c], table_p)
        outs.append(_tc_head(emb_c, wcat, cb, W1, b1r, W2, b2r))
    return jnp.concatenate(outs, axis=0)
